# baseline (speedup 1.0000x reference)
import functools

import jax
import jax.numpy as jnp
from jax import lax
from jax.experimental import pallas as pl
from jax.experimental.pallas import tpu as pltpu
from jax.experimental.pallas import tpu_sc as plsc

N = 10000
E = 320000
IN_D = 128
HID = 16
OUT_D = 4
NW8 = N // 8

NC, NS = 2, 16
NW = NC * NS
E_PER_W = E // NW
CHUNK = 80
NCHUNK = E_PER_W // CHUNK
RPS = N // NS
_MESH = plsc.VectorSubcoreMesh(core_axis_name="c", subcore_axis_name="s",
                               num_cores=NC, num_subcores=NS)
_SC_PARAMS = pltpu.CompilerParams(use_tc_tiling_on_sc=False)


MSG_NBUF = 12
MSG_NFULL = NCHUNK // MSG_NBUF
MSG_NREM = NCHUNK % MSG_NBUF
DEG_NBUF = 16
DEG_NFULL = NCHUNK // DEG_NBUF
DEG_NREM = NCHUNK % DEG_NBUF


@functools.partial(
    pl.kernel,
    out_type=jax.ShapeDtypeStruct((NC, N, HID), jnp.float32),
    mesh=_MESH,
    scratch_types=[
        pltpu.VMEM((E_PER_W,), jnp.int32),
        pltpu.VMEM((E_PER_W,), jnp.int32),
        [pltpu.VMEM((CHUNK, HID), jnp.float32) for _ in range(MSG_NBUF)],
        pltpu.VMEM((RPS, HID), jnp.float32),
        pltpu.VMEM_SHARED((N, HID), jnp.float32),
        pltpu.SemaphoreType.DMA((MSG_NBUF,)),
        pltpu.SemaphoreType.DMA((MSG_NBUF,)),
    ],
    compiler_params=_SC_PARAMS,
)
def _msg_pass(table_hbm, ei_hbm, zeros_hbm, out_hbm,
              sidx_all, didx_all, rows, stg, acc_sh, sem_g, sem_s):
    c = lax.axis_index("c")
    s = lax.axis_index("s")
    wid = c * NS + s
    base = wid * E_PER_W
    pltpu.sync_copy(zeros_hbm, stg)
    pltpu.sync_copy(stg, acc_sh.at[pl.ds(s * RPS, RPS)])
    pltpu.sync_copy(ei_hbm.at[0, pl.ds(base, E_PER_W)], sidx_all)
    pltpu.sync_copy(ei_hbm.at[1, pl.ds(base, E_PER_W)], didx_all)
    plsc.subcore_barrier()

    def gather_of(j, b):
        return pltpu.make_async_copy(
            table_hbm.at[sidx_all.at[pl.ds(j * CHUNK, CHUNK)]],
            rows[b], sem_g.at[b])

    def scat_of(j, b):
        return pltpu.make_async_copy(
            rows[b], acc_sh.at[didx_all.at[pl.ds(j * CHUNK, CHUNK)]],
            sem_s.at[b])

    def fire(j, b):
        gather_of(j, b).wait()
        pltpu.async_copy(
            rows[b], acc_sh.at[didx_all.at[pl.ds(j * CHUNK, CHUNK)]],
            sem_s.at[b], add=True)

    for b in range(MSG_NBUF):
        gather_of(b, b).start()

    def outer(g, carry):
        j0 = g * MSG_NBUF
        for b in range(MSG_NBUF):
            fire(j0 + b, b)
        for b in range(MSG_NBUF):
            scat_of(j0 + b, b).wait()
            gather_of(j0 + MSG_NBUF + b, b).start()
        return carry
    lax.fori_loop(0, MSG_NFULL - 1, outer, 0)

    j0 = (MSG_NFULL - 1) * MSG_NBUF
    for b in range(MSG_NBUF):
        fire(j0 + b, b)
    for b in range(MSG_NREM):
        scat_of(j0 + b, b).wait()
        gather_of(MSG_NFULL * MSG_NBUF + b, b).start()
    for b in range(MSG_NREM):
        fire(MSG_NFULL * MSG_NBUF + b, b)
    for b in range(MSG_NREM):
        scat_of(MSG_NFULL * MSG_NBUF + b, b).wait()
    for b in range(MSG_NREM, MSG_NBUF):
        scat_of(j0 + b, b).wait()

    plsc.subcore_barrier()
    pltpu.sync_copy(acc_sh.at[pl.ds(s * RPS, RPS)], stg)
    pltpu.sync_copy(stg, out_hbm.at[c, pl.ds(s * RPS, RPS)])


@functools.partial(
    pl.kernel,
    out_type=jax.ShapeDtypeStruct((NC, N, HID), jnp.float32),
    mesh=_MESH,
    scratch_types=[
        pltpu.VMEM((E_PER_W,), jnp.int32),
        pltpu.VMEM((CHUNK, HID), jnp.float32),
        pltpu.VMEM((RPS, HID), jnp.float32),
        pltpu.VMEM_SHARED((N, HID), jnp.float32),
        pltpu.SemaphoreType.DMA((DEG_NBUF,)),
    ],
    compiler_params=_SC_PARAMS,
)
def _deg_pass(ei_hbm, zeros_hbm, ones_hbm, out_hbm,
              didx_all, ones, stg, acc_sh, sem_s):
    c = lax.axis_index("c")
    s = lax.axis_index("s")
    wid = c * NS + s
    base = wid * E_PER_W
    pltpu.sync_copy(zeros_hbm, stg)
    pltpu.sync_copy(stg, acc_sh.at[pl.ds(s * RPS, RPS)])
    pltpu.sync_copy(ones_hbm, ones)
    pltpu.sync_copy(ei_hbm.at[1, pl.ds(base, E_PER_W)], didx_all)
    plsc.subcore_barrier()

    def scat_of(j, b):
        return pltpu.make_async_copy(
            ones, acc_sh.at[didx_all.at[pl.ds(j * CHUNK, CHUNK)]],
            sem_s.at[b])

    def fire(j, b):
        pltpu.async_copy(
            ones, acc_sh.at[didx_all.at[pl.ds(j * CHUNK, CHUNK)]],
            sem_s.at[b], add=True)

    for b in range(DEG_NBUF):
        fire(b, b)

    def outer(g, carry):
        j0 = g * DEG_NBUF
        for b in range(DEG_NBUF):
            scat_of(j0 + b, b).wait()
            fire(j0 + DEG_NBUF + b, b)
        return carry
    lax.fori_loop(0, DEG_NFULL - 1, outer, 0)

    j0 = (DEG_NFULL - 1) * DEG_NBUF
    for b in range(DEG_NREM):
        scat_of(j0 + b, b).wait()
        fire(DEG_NFULL * DEG_NBUF + b, b)
    for b in range(DEG_NREM):
        scat_of(DEG_NFULL * DEG_NBUF + b, b).wait()
    for b in range(DEG_NREM, DEG_NBUF):
        scat_of(j0 + b, b).wait()

    plsc.subcore_barrier()
    pltpu.sync_copy(acc_sh.at[pl.ds(s * RPS, RPS)], stg)
    pltpu.sync_copy(stg, out_hbm.at[c, pl.ds(s * RPS, RPS)])


def _dinv_wide(dacc_ref):
    return lax.rsqrt(dacc_ref[0] + dacc_ref[1] + 1.0)


def _tca_body(x_ref, w1_ref, dacc_ref, g1_ref):
    w1 = w1_ref[...]
    parts = [
        jnp.dot(x_ref[:, k, :], w1, preferred_element_type=jnp.float32)
        for k in range(8)
    ]
    g1_ref[...] = jnp.concatenate(parts, axis=1) * _dinv_wide(dacc_ref)


def _tca(x, W1, dacc_w):
    return pl.pallas_call(
        _tca_body,
        out_shape=jax.ShapeDtypeStruct((NW8, 128), jnp.float32),
    )(x.reshape(NW8, 8, IN_D), W1, dacc_w)


def _tcb_body(acc_ref, g1_ref, dacc_ref, b1_ref, g2_ref):
    dinv = _dinv_wide(dacc_ref)
    z = jnp.maximum(dinv * (acc_ref[0] + acc_ref[1] + g1_ref[...])
                    + b1_ref[...], 0.0)
    g2_ref[...] = dinv * z


def _tcb(acc1_w, g1_w, dacc_w, b1):
    return pl.pallas_call(
        _tcb_body,
        out_shape=jax.ShapeDtypeStruct((NW8, 128), jnp.float32),
    )(acc1_w, g1_w, dacc_w, jnp.tile(b1, 8).reshape(1, 128))


def _tcc_body(acc_ref, g2_ref, dacc_ref, w2b_ref, b2w_ref, savg_ref,
              ssum_ref, out_ref):
    dinv = _dinv_wide(dacc_ref)
    t_w = dinv * (acc_ref[0] + acc_ref[1] + g2_ref[...])
    h = jnp.dot(t_w, w2b_ref[...], preferred_element_type=jnp.float32)
    h = h + b2w_ref[...]
    m = jnp.dot(h, savg_ref[...], preferred_element_type=jnp.float32)
    e = jnp.exp(h - m)
    lse = jnp.log(jnp.dot(e, ssum_ref[...],
                          preferred_element_type=jnp.float32))
    out_ref[...] = h - m - lse


def _tcc(acc2_w, g2_w, dacc_w, W2, b2):
    eye8 = jnp.eye(8, dtype=jnp.float32)
    w2big = jnp.kron(eye8, W2)
    savg = jnp.kron(eye8, jnp.full((OUT_D, OUT_D), 0.25, jnp.float32))
    ssum = jnp.kron(eye8, jnp.ones((OUT_D, OUT_D), jnp.float32))
    b2w = jnp.tile(b2, 8).reshape(1, 8 * OUT_D)
    out_w = pl.pallas_call(
        _tcc_body,
        out_shape=jax.ShapeDtypeStruct((NW8, 8 * OUT_D), jnp.float32),
    )(acc2_w, g2_w, dacc_w, w2big, b2w, savg, ssum)
    return out_w.reshape(N, OUT_D)


def kernel(x, edge_index, W1, b1, W2, b2):
    ei = edge_index.astype(jnp.int32)
    zeros_c = jnp.zeros((RPS, HID), jnp.float32)
    ones_c = jnp.ones((CHUNK, HID), jnp.float32)
    dacc_w = _deg_pass(ei, zeros_c, ones_c).reshape(NC, NW8, 128)
    g1_w = _tca(x, W1, dacc_w)
    acc1_w = _msg_pass(g1_w.reshape(N, HID), ei, zeros_c).reshape(NC, NW8, 128)
    g2_w = _tcb(acc1_w, g1_w, dacc_w, b1)
    acc2_w = _msg_pass(g2_w.reshape(N, HID), ei, zeros_c).reshape(NC, NW8, 128)
    return _tcc(acc2_w, g2_w, dacc_w, W2, b2)


# speedup vs baseline: 1.0252x; 1.0252x over previous
import functools

import jax
import jax.numpy as jnp
from jax import lax
from jax.experimental import pallas as pl
from jax.experimental.pallas import tpu as pltpu
from jax.experimental.pallas import tpu_sc as plsc

N = 10000
E = 320000
IN_D = 128
HID = 16
OUT_D = 4
NW8 = N // 8

NC, NS = 2, 16
NW = NC * NS
E_PER_W = E // NW
CHUNK = 80
NCHUNK = E_PER_W // CHUNK
RPS = N // NS
_MESH = plsc.VectorSubcoreMesh(core_axis_name="c", subcore_axis_name="s",
                               num_cores=NC, num_subcores=NS)
_SC_PARAMS = pltpu.CompilerParams(use_tc_tiling_on_sc=False)


MSG_NBUF = 12
MSG_NFULL = NCHUNK // MSG_NBUF
MSG_NREM = NCHUNK % MSG_NBUF
DEG_NBUF = 16
DEG_NFULL = NCHUNK // DEG_NBUF
DEG_NREM = NCHUNK % DEG_NBUF


@functools.partial(
    pl.kernel,
    out_type=jax.ShapeDtypeStruct((NC, N, HID), jnp.float32),
    mesh=_MESH,
    scratch_types=[
        pltpu.VMEM((E_PER_W,), jnp.int32),
        pltpu.VMEM((E_PER_W,), jnp.int32),
        [pltpu.VMEM((CHUNK, HID), jnp.float32) for _ in range(MSG_NBUF)],
        pltpu.VMEM((RPS, HID), jnp.float32),
        pltpu.VMEM_SHARED((N, HID), jnp.float32),
        pltpu.SemaphoreType.DMA((MSG_NBUF,)),
        pltpu.SemaphoreType.DMA((MSG_NBUF,)),
    ],
    compiler_params=_SC_PARAMS,
)
def _msg_pass(table_hbm, ei_hbm, zeros_hbm, out_hbm,
              sidx_all, didx_all, rows, stg, acc_sh, sem_g, sem_s):
    c = lax.axis_index("c")
    s = lax.axis_index("s")
    wid = c * NS + s
    base = wid * E_PER_W
    pltpu.sync_copy(zeros_hbm, stg)
    pltpu.sync_copy(stg, acc_sh.at[pl.ds(s * RPS, RPS)])
    pltpu.sync_copy(ei_hbm.at[0, pl.ds(base, E_PER_W)], sidx_all)
    pltpu.sync_copy(ei_hbm.at[1, pl.ds(base, E_PER_W)], didx_all)
    plsc.subcore_barrier()

    def gather_of(j, b):
        return pltpu.make_async_copy(
            table_hbm.at[sidx_all.at[pl.ds(j * CHUNK, CHUNK)]],
            rows[b], sem_g.at[b])

    def scat_of(j, b):
        return pltpu.make_async_copy(
            rows[b], acc_sh.at[didx_all.at[pl.ds(j * CHUNK, CHUNK)]],
            sem_s.at[b])

    def fire(j, b):
        gather_of(j, b).wait()
        pltpu.async_copy(
            rows[b], acc_sh.at[didx_all.at[pl.ds(j * CHUNK, CHUNK)]],
            sem_s.at[b], add=True)

    for b in range(MSG_NBUF):
        gather_of(b, b).start()

    def outer(g, carry):
        j0 = g * MSG_NBUF
        for b in range(MSG_NBUF):
            fire(j0 + b, b)
        for b in range(MSG_NBUF):
            scat_of(j0 + b, b).wait()
            gather_of(j0 + MSG_NBUF + b, b).start()
        return carry
    lax.fori_loop(0, MSG_NFULL - 1, outer, 0)

    j0 = (MSG_NFULL - 1) * MSG_NBUF
    for b in range(MSG_NBUF):
        fire(j0 + b, b)
    for b in range(MSG_NREM):
        scat_of(j0 + b, b).wait()
        gather_of(MSG_NFULL * MSG_NBUF + b, b).start()
    for b in range(MSG_NREM):
        fire(MSG_NFULL * MSG_NBUF + b, b)
    for b in range(MSG_NREM):
        scat_of(MSG_NFULL * MSG_NBUF + b, b).wait()
    for b in range(MSG_NREM, MSG_NBUF):
        scat_of(j0 + b, b).wait()

    plsc.subcore_barrier()
    pltpu.sync_copy(acc_sh.at[pl.ds(s * RPS, RPS)], stg)
    pltpu.sync_copy(stg, out_hbm.at[c, pl.ds(s * RPS, RPS)])


@functools.partial(
    pl.kernel,
    out_type=jax.ShapeDtypeStruct((NC, N, HID), jnp.float32),
    mesh=_MESH,
    scratch_types=[
        pltpu.VMEM((E_PER_W,), jnp.int32),
        pltpu.VMEM((CHUNK, HID), jnp.float32),
        pltpu.VMEM((RPS, HID), jnp.float32),
        pltpu.VMEM_SHARED((N, HID), jnp.float32),
        pltpu.SemaphoreType.DMA((DEG_NBUF,)),
    ],
    compiler_params=_SC_PARAMS,
)
def _deg_pass(ei_hbm, zeros_hbm, ones_hbm, out_hbm,
              didx_all, ones, stg, acc_sh, sem_s):
    c = lax.axis_index("c")
    s = lax.axis_index("s")
    wid = c * NS + s
    base = wid * E_PER_W
    pltpu.sync_copy(zeros_hbm, stg)
    pltpu.sync_copy(stg, acc_sh.at[pl.ds(s * RPS, RPS)])
    pltpu.sync_copy(ones_hbm, ones)
    pltpu.sync_copy(ei_hbm.at[1, pl.ds(base, E_PER_W)], didx_all)
    plsc.subcore_barrier()

    def scat_of(j, b):
        return pltpu.make_async_copy(
            ones, acc_sh.at[didx_all.at[pl.ds(j * CHUNK, CHUNK)]],
            sem_s.at[b])

    def fire(j, b):
        pltpu.async_copy(
            ones, acc_sh.at[didx_all.at[pl.ds(j * CHUNK, CHUNK)]],
            sem_s.at[b], add=True)

    for b in range(DEG_NBUF):
        fire(b, b)

    def outer(g, carry):
        j0 = g * DEG_NBUF
        for b in range(DEG_NBUF):
            scat_of(j0 + b, b).wait()
            fire(j0 + DEG_NBUF + b, b)
        return carry
    lax.fori_loop(0, DEG_NFULL - 1, outer, 0)

    j0 = (DEG_NFULL - 1) * DEG_NBUF
    for b in range(DEG_NREM):
        scat_of(j0 + b, b).wait()
        fire(DEG_NFULL * DEG_NBUF + b, b)
    for b in range(DEG_NREM):
        scat_of(DEG_NFULL * DEG_NBUF + b, b).wait()
    for b in range(DEG_NREM, DEG_NBUF):
        scat_of(j0 + b, b).wait()

    plsc.subcore_barrier()
    pltpu.sync_copy(acc_sh.at[pl.ds(s * RPS, RPS)], stg)
    pltpu.sync_copy(stg, out_hbm.at[c, pl.ds(s * RPS, RPS)])


def _dinv_wide(dacc_ref):
    return lax.rsqrt(dacc_ref[0] + dacc_ref[1] + 1.0)


def _tcmm_body(x_ref, w1_ref, h_ref):
    w1 = w1_ref[...]
    parts = [
        jnp.dot(x_ref[:, k, :], w1, preferred_element_type=jnp.float32)
        for k in range(8)
    ]
    h_ref[...] = jnp.concatenate(parts, axis=1)


def _tcmm(x, W1):
    return pl.pallas_call(
        _tcmm_body,
        out_shape=jax.ShapeDtypeStruct((NW8, 128), jnp.float32),
    )(x.reshape(NW8, 8, IN_D), W1)


def _tca_body(h_ref, dacc_ref, g1_ref):
    g1_ref[...] = h_ref[...] * _dinv_wide(dacc_ref)


def _tca(h_w, dacc_w):
    return pl.pallas_call(
        _tca_body,
        out_shape=jax.ShapeDtypeStruct((NW8, 128), jnp.float32),
    )(h_w, dacc_w)


def _tcb_body(acc_ref, g1_ref, dacc_ref, b1_ref, g2_ref):
    dinv = _dinv_wide(dacc_ref)
    z = jnp.maximum(dinv * (acc_ref[0] + acc_ref[1] + g1_ref[...])
                    + b1_ref[...], 0.0)
    g2_ref[...] = dinv * z


def _tcb(acc1_w, g1_w, dacc_w, b1):
    return pl.pallas_call(
        _tcb_body,
        out_shape=jax.ShapeDtypeStruct((NW8, 128), jnp.float32),
    )(acc1_w, g1_w, dacc_w, jnp.tile(b1, 8).reshape(1, 128))


def _tcc_body(acc_ref, g2_ref, dacc_ref, w2b_ref, b2w_ref, savg_ref,
              ssum_ref, out_ref):
    dinv = _dinv_wide(dacc_ref)
    t_w = dinv * (acc_ref[0] + acc_ref[1] + g2_ref[...])
    h = jnp.dot(t_w, w2b_ref[...], preferred_element_type=jnp.float32)
    h = h + b2w_ref[...]
    m = jnp.dot(h, savg_ref[...], preferred_element_type=jnp.float32)
    e = jnp.exp(h - m)
    lse = jnp.log(jnp.dot(e, ssum_ref[...],
                          preferred_element_type=jnp.float32))
    out_ref[...] = h - m - lse


def _tcc(acc2_w, g2_w, dacc_w, W2, b2):
    eye8 = jnp.eye(8, dtype=jnp.float32)
    w2big = jnp.kron(eye8, W2)
    savg = jnp.kron(eye8, jnp.full((OUT_D, OUT_D), 0.25, jnp.float32))
    ssum = jnp.kron(eye8, jnp.ones((OUT_D, OUT_D), jnp.float32))
    b2w = jnp.tile(b2, 8).reshape(1, 8 * OUT_D)
    out_w = pl.pallas_call(
        _tcc_body,
        out_shape=jax.ShapeDtypeStruct((NW8, 8 * OUT_D), jnp.float32),
    )(acc2_w, g2_w, dacc_w, w2big, b2w, savg, ssum)
    return out_w.reshape(N, OUT_D)


def kernel(x, edge_index, W1, b1, W2, b2):
    ei = edge_index.astype(jnp.int32)
    zeros_c = jnp.zeros((RPS, HID), jnp.float32)
    ones_c = jnp.ones((CHUNK, HID), jnp.float32)
    dacc_w = _deg_pass(ei, zeros_c, ones_c).reshape(NC, NW8, 128)
    h_w = _tcmm(x, W1)
    g1_w = _tca(h_w, dacc_w)
    acc1_w = _msg_pass(g1_w.reshape(N, HID), ei, zeros_c).reshape(NC, NW8, 128)
    g2_w = _tcb(acc1_w, g1_w, dacc_w, b1)
    acc2_w = _msg_pass(g2_w.reshape(N, HID), ei, zeros_c).reshape(NC, NW8, 128)
    return _tcc(acc2_w, g2_w, dacc_w, W2, b2)


# speedup vs baseline: 1.0703x; 1.0440x over previous
import functools

import jax
import jax.numpy as jnp
from jax import lax
from jax.experimental import pallas as pl
from jax.experimental.pallas import tpu as pltpu
from jax.experimental.pallas import tpu_sc as plsc

N = 10000
E = 320000
IN_D = 128
HID = 16
OUT_D = 4
NW8 = N // 8

NC, NS = 2, 16
NW = NC * NS
E_PER_W = E // NW
CHUNK = 80
NCHUNK = E_PER_W // CHUNK
RPS = N // NS
_MESH = plsc.VectorSubcoreMesh(core_axis_name="c", subcore_axis_name="s",
                               num_cores=NC, num_subcores=NS)
_SC_PARAMS = pltpu.CompilerParams(use_tc_tiling_on_sc=False)


MSG_NBUF = 12
MSG_NFULL = NCHUNK // MSG_NBUF
MSG_NREM = NCHUNK % MSG_NBUF
DEG_W = 8
DEG_NBUF = 16
DEG_NFULL = NCHUNK // DEG_NBUF
DEG_NREM = NCHUNK % DEG_NBUF


@functools.partial(
    pl.kernel,
    out_type=jax.ShapeDtypeStruct((NC, N, HID), jnp.float32),
    mesh=_MESH,
    scratch_types=[
        pltpu.VMEM((E_PER_W,), jnp.int32),
        pltpu.VMEM((E_PER_W,), jnp.int32),
        [pltpu.VMEM((CHUNK, HID), jnp.float32) for _ in range(MSG_NBUF)],
        pltpu.VMEM((RPS, HID), jnp.float32),
        pltpu.VMEM_SHARED((N, HID), jnp.float32),
        pltpu.SemaphoreType.DMA((MSG_NBUF,)),
        pltpu.SemaphoreType.DMA((MSG_NBUF,)),
    ],
    compiler_params=_SC_PARAMS,
)
def _msg_pass(table_hbm, ei_hbm, zeros_hbm, out_hbm,
              sidx_all, didx_all, rows, stg, acc_sh, sem_g, sem_s):
    c = lax.axis_index("c")
    s = lax.axis_index("s")
    wid = c * NS + s
    base = wid * E_PER_W
    pltpu.sync_copy(zeros_hbm, stg)
    pltpu.sync_copy(stg, acc_sh.at[pl.ds(s * RPS, RPS)])
    pltpu.sync_copy(ei_hbm.at[0, pl.ds(base, E_PER_W)], sidx_all)
    pltpu.sync_copy(ei_hbm.at[1, pl.ds(base, E_PER_W)], didx_all)
    plsc.subcore_barrier()

    def gather_of(j, b):
        return pltpu.make_async_copy(
            table_hbm.at[sidx_all.at[pl.ds(j * CHUNK, CHUNK)]],
            rows[b], sem_g.at[b])

    def scat_of(j, b):
        return pltpu.make_async_copy(
            rows[b], acc_sh.at[didx_all.at[pl.ds(j * CHUNK, CHUNK)]],
            sem_s.at[b])

    def fire(j, b):
        gather_of(j, b).wait()
        pltpu.async_copy(
            rows[b], acc_sh.at[didx_all.at[pl.ds(j * CHUNK, CHUNK)]],
            sem_s.at[b], add=True)

    for b in range(MSG_NBUF):
        gather_of(b, b).start()

    def outer(g, carry):
        j0 = g * MSG_NBUF
        for b in range(MSG_NBUF):
            fire(j0 + b, b)
        for b in range(MSG_NBUF):
            scat_of(j0 + b, b).wait()
            gather_of(j0 + MSG_NBUF + b, b).start()
        return carry
    lax.fori_loop(0, MSG_NFULL - 1, outer, 0)

    j0 = (MSG_NFULL - 1) * MSG_NBUF
    for b in range(MSG_NBUF):
        fire(j0 + b, b)
    for b in range(MSG_NREM):
        scat_of(j0 + b, b).wait()
        gather_of(MSG_NFULL * MSG_NBUF + b, b).start()
    for b in range(MSG_NREM):
        fire(MSG_NFULL * MSG_NBUF + b, b)
    for b in range(MSG_NREM):
        scat_of(MSG_NFULL * MSG_NBUF + b, b).wait()
    for b in range(MSG_NREM, MSG_NBUF):
        scat_of(j0 + b, b).wait()

    plsc.subcore_barrier()
    pltpu.sync_copy(acc_sh.at[pl.ds(s * RPS, RPS)], stg)
    pltpu.sync_copy(stg, out_hbm.at[c, pl.ds(s * RPS, RPS)])


@functools.partial(
    pl.kernel,
    out_type=jax.ShapeDtypeStruct((NC, N, DEG_W), jnp.float32),
    mesh=_MESH,
    scratch_types=[
        pltpu.VMEM((E_PER_W,), jnp.int32),
        pltpu.VMEM((CHUNK, DEG_W), jnp.float32),
        pltpu.VMEM((RPS, DEG_W), jnp.float32),
        pltpu.VMEM_SHARED((N, DEG_W), jnp.float32),
        pltpu.SemaphoreType.DMA((DEG_NBUF,)),
    ],
    compiler_params=_SC_PARAMS,
)
def _deg_pass(ei_hbm, zeros_hbm, ones_hbm, out_hbm,
              didx_all, ones, stg, acc_sh, sem_s):
    c = lax.axis_index("c")
    s = lax.axis_index("s")
    wid = c * NS + s
    base = wid * E_PER_W
    pltpu.sync_copy(zeros_hbm, stg)
    pltpu.sync_copy(stg, acc_sh.at[pl.ds(s * RPS, RPS)])
    pltpu.sync_copy(ones_hbm, ones)
    pltpu.sync_copy(ei_hbm.at[1, pl.ds(base, E_PER_W)], didx_all)
    plsc.subcore_barrier()

    def scat_of(j, b):
        return pltpu.make_async_copy(
            ones, acc_sh.at[didx_all.at[pl.ds(j * CHUNK, CHUNK)]],
            sem_s.at[b])

    def fire(j, b):
        pltpu.async_copy(
            ones, acc_sh.at[didx_all.at[pl.ds(j * CHUNK, CHUNK)]],
            sem_s.at[b], add=True)

    for b in range(DEG_NBUF):
        fire(b, b)

    def outer(g, carry):
        j0 = g * DEG_NBUF
        for b in range(DEG_NBUF):
            scat_of(j0 + b, b).wait()
            fire(j0 + DEG_NBUF + b, b)
        return carry
    lax.fori_loop(0, DEG_NFULL - 1, outer, 0)

    j0 = (DEG_NFULL - 1) * DEG_NBUF
    for b in range(DEG_NREM):
        scat_of(j0 + b, b).wait()
        fire(DEG_NFULL * DEG_NBUF + b, b)
    for b in range(DEG_NREM):
        scat_of(DEG_NFULL * DEG_NBUF + b, b).wait()
    for b in range(DEG_NREM, DEG_NBUF):
        scat_of(j0 + b, b).wait()

    plsc.subcore_barrier()
    pltpu.sync_copy(acc_sh.at[pl.ds(s * RPS, RPS)], stg)
    pltpu.sync_copy(stg, out_hbm.at[c, pl.ds(s * RPS, RPS)])


def _lane_expand_consts():
    a = jnp.arange(128)[:, None]
    c = jnp.arange(128)[None, :]
    pe = (a == (c // HID) * DEG_W).astype(jnp.float32)
    po = (a == (c // HID) * DEG_W + 64).astype(jnp.float32)
    return pe, po


def _tcmm_body(x_ref, w1_ref, h_ref):
    w1 = w1_ref[...]
    parts = [
        jnp.dot(x_ref[:, k, :], w1, preferred_element_type=jnp.float32)
        for k in range(8)
    ]
    h_ref[...] = jnp.concatenate(parts, axis=1)


def _tcmm(x, W1):
    return pl.pallas_call(
        _tcmm_body,
        out_shape=jax.ShapeDtypeStruct((NW8, 128), jnp.float32),
    )(x.reshape(NW8, 8, IN_D), W1)


def _tca_body(h_ref, d8_ref, pe_ref, po_ref, g1_ref, dinv_ref):
    deg8 = d8_ref[0] + d8_ref[1] + 1.0
    dup = jnp.reshape(
        jnp.broadcast_to(deg8[:, None, :], (N // 16, 2, 128)), (NW8, 128))
    e = jnp.dot(dup, pe_ref[...], preferred_element_type=jnp.float32)
    o = jnp.dot(dup, po_ref[...], preferred_element_type=jnp.float32)
    par = (lax.broadcasted_iota(jnp.int32, (NW8, 128), 0) % 2) == 1
    dinv = lax.rsqrt(jnp.where(par, o, e))
    dinv_ref[...] = dinv
    g1_ref[...] = h_ref[...] * dinv


def _tca(h_w, deg8_w):
    pe, po = _lane_expand_consts()
    return pl.pallas_call(
        _tca_body,
        out_shape=[jax.ShapeDtypeStruct((NW8, 128), jnp.float32),
                   jax.ShapeDtypeStruct((NW8, 128), jnp.float32)],
    )(h_w, deg8_w, pe, po)


def _tcb_body(acc_ref, g1_ref, dinv_ref, b1_ref, g2_ref):
    dinv = dinv_ref[...]
    z = jnp.maximum(dinv * (acc_ref[0] + acc_ref[1] + g1_ref[...])
                    + b1_ref[...], 0.0)
    g2_ref[...] = dinv * z


def _tcb(acc1_w, g1_w, dinv_w, b1):
    return pl.pallas_call(
        _tcb_body,
        out_shape=jax.ShapeDtypeStruct((NW8, 128), jnp.float32),
    )(acc1_w, g1_w, dinv_w, jnp.tile(b1, 8).reshape(1, 128))


def _tcc_body(acc_ref, g2_ref, dinv_ref, w2b_ref, b2w_ref, savg_ref,
              ssum_ref, out_ref):
    t_w = dinv_ref[...] * (acc_ref[0] + acc_ref[1] + g2_ref[...])
    h = jnp.dot(t_w, w2b_ref[...], preferred_element_type=jnp.float32)
    h = h + b2w_ref[...]
    m = jnp.dot(h, savg_ref[...], preferred_element_type=jnp.float32)
    e = jnp.exp(h - m)
    lse = jnp.log(jnp.dot(e, ssum_ref[...],
                          preferred_element_type=jnp.float32))
    out_ref[...] = h - m - lse


def _tcc(acc2_w, g2_w, dinv_w, W2, b2):
    eye8 = jnp.eye(8, dtype=jnp.float32)
    w2big = jnp.kron(eye8, W2)
    savg = jnp.kron(eye8, jnp.full((OUT_D, OUT_D), 0.25, jnp.float32))
    ssum = jnp.kron(eye8, jnp.ones((OUT_D, OUT_D), jnp.float32))
    b2w = jnp.tile(b2, 8).reshape(1, 8 * OUT_D)
    out_w = pl.pallas_call(
        _tcc_body,
        out_shape=jax.ShapeDtypeStruct((NW8, 8 * OUT_D), jnp.float32),
    )(acc2_w, g2_w, dinv_w, w2big, b2w, savg, ssum)
    return out_w.reshape(N, OUT_D)


def kernel(x, edge_index, W1, b1, W2, b2):
    ei = edge_index.astype(jnp.int32)
    zeros_c = jnp.zeros((RPS, HID), jnp.float32)
    zeros8_c = jnp.zeros((RPS, DEG_W), jnp.float32)
    ones8_c = jnp.ones((CHUNK, DEG_W), jnp.float32)
    deg8_w = _deg_pass(ei, zeros8_c, ones8_c).reshape(NC, N // 16, 128)
    h_w = _tcmm(x, W1)
    g1_w, dinv_w = _tca(h_w, deg8_w)
    acc1_w = _msg_pass(g1_w.reshape(N, HID), ei, zeros_c).reshape(NC, NW8, 128)
    g2_w = _tcb(acc1_w, g1_w, dinv_w, b1)
    acc2_w = _msg_pass(g2_w.reshape(N, HID), ei, zeros_c).reshape(NC, NW8, 128)
    return _tcc(acc2_w, g2_w, dinv_w, W2, b2)


# speedup vs baseline: 1.1075x; 1.0348x over previous
import functools

import jax
import jax.numpy as jnp
from jax import lax
from jax.experimental import pallas as pl
from jax.experimental.pallas import tpu as pltpu
from jax.experimental.pallas import tpu_sc as plsc

N = 10000
E = 320000
IN_D = 128
HID = 16
OUT_D = 4
NW8 = N // 8

NC, NS = 2, 16
NW = NC * NS
E_PER_W = E // NW
CHUNK = 80
NCHUNK = E_PER_W // CHUNK
RPS = N // NS
_MESH = plsc.VectorSubcoreMesh(core_axis_name="c", subcore_axis_name="s",
                               num_cores=NC, num_subcores=NS)
_SC_PARAMS = pltpu.CompilerParams(use_tc_tiling_on_sc=False)


MSG_NBUF = 12
MSG_NFULL = NCHUNK // MSG_NBUF
MSG_NREM = NCHUNK % MSG_NBUF
DEG_W = 8
DEG_NBUF = 16
DEG_NFULL = NCHUNK // DEG_NBUF
DEG_NREM = NCHUNK % DEG_NBUF


@functools.partial(
    pl.kernel,
    out_type=jax.ShapeDtypeStruct((NC, N, HID), jnp.float32),
    mesh=_MESH,
    scratch_types=[
        pltpu.VMEM((E_PER_W,), jnp.int32),
        pltpu.VMEM((E_PER_W,), jnp.int32),
        [pltpu.VMEM((CHUNK, HID), jnp.float32) for _ in range(MSG_NBUF)],
        pltpu.VMEM((RPS, HID), jnp.float32),
        pltpu.VMEM_SHARED((N, HID), jnp.float32),
        pltpu.SemaphoreType.DMA((MSG_NBUF,)),
        pltpu.SemaphoreType.DMA((MSG_NBUF,)),
    ],
    compiler_params=_SC_PARAMS,
)
def _msg_pass(table_hbm, ei_hbm, zeros_hbm, out_hbm,
              sidx_all, didx_all, rows, stg, acc_sh, sem_g, sem_s):
    c = lax.axis_index("c")
    s = lax.axis_index("s")
    wid = c * NS + s
    base = wid * E_PER_W
    ld_s = pltpu.make_async_copy(
        ei_hbm.at[0, pl.ds(base, E_PER_W)], sidx_all, sem_g.at[0])
    ld_d = pltpu.make_async_copy(
        ei_hbm.at[1, pl.ds(base, E_PER_W)], didx_all, sem_g.at[1])
    ld_s.start()
    ld_d.start()
    pltpu.sync_copy(zeros_hbm, stg)
    pltpu.sync_copy(stg, acc_sh.at[pl.ds(s * RPS, RPS)])
    ld_s.wait()
    ld_d.wait()
    plsc.subcore_barrier()

    def gather_of(j, b):
        return pltpu.make_async_copy(
            table_hbm.at[sidx_all.at[pl.ds(j * CHUNK, CHUNK)]],
            rows[b], sem_g.at[b])

    def scat_of(j, b):
        return pltpu.make_async_copy(
            rows[b], acc_sh.at[didx_all.at[pl.ds(j * CHUNK, CHUNK)]],
            sem_s.at[b])

    def fire(j, b):
        gather_of(j, b).wait()
        pltpu.async_copy(
            rows[b], acc_sh.at[didx_all.at[pl.ds(j * CHUNK, CHUNK)]],
            sem_s.at[b], add=True)

    for b in range(MSG_NBUF):
        gather_of(b, b).start()

    def outer(g, carry):
        j0 = g * MSG_NBUF
        for b in range(MSG_NBUF):
            fire(j0 + b, b)
        for b in range(MSG_NBUF):
            scat_of(j0 + b, b).wait()
            gather_of(j0 + MSG_NBUF + b, b).start()
        return carry
    lax.fori_loop(0, MSG_NFULL - 1, outer, 0)

    j0 = (MSG_NFULL - 1) * MSG_NBUF
    for b in range(MSG_NBUF):
        fire(j0 + b, b)
    for b in range(MSG_NREM):
        scat_of(j0 + b, b).wait()
        gather_of(MSG_NFULL * MSG_NBUF + b, b).start()
    for b in range(MSG_NREM):
        fire(MSG_NFULL * MSG_NBUF + b, b)
    for b in range(MSG_NREM):
        scat_of(MSG_NFULL * MSG_NBUF + b, b).wait()
    for b in range(MSG_NREM, MSG_NBUF):
        scat_of(j0 + b, b).wait()

    plsc.subcore_barrier()
    pltpu.sync_copy(acc_sh.at[pl.ds(s * RPS, RPS)], stg)
    pltpu.sync_copy(stg, out_hbm.at[c, pl.ds(s * RPS, RPS)])


@functools.partial(
    pl.kernel,
    out_type=jax.ShapeDtypeStruct((NC, N, DEG_W), jnp.float32),
    mesh=_MESH,
    scratch_types=[
        pltpu.VMEM((E_PER_W,), jnp.int32),
        pltpu.VMEM((CHUNK, DEG_W), jnp.float32),
        pltpu.VMEM((RPS, DEG_W), jnp.float32),
        pltpu.VMEM_SHARED((N, DEG_W), jnp.float32),
        pltpu.SemaphoreType.DMA((DEG_NBUF,)),
    ],
    compiler_params=_SC_PARAMS,
)
def _deg_pass(ei_hbm, zeros_hbm, ones_hbm, out_hbm,
              didx_all, ones, stg, acc_sh, sem_s):
    c = lax.axis_index("c")
    s = lax.axis_index("s")
    wid = c * NS + s
    base = wid * E_PER_W
    ld_o = pltpu.make_async_copy(ones_hbm, ones, sem_s.at[0])
    ld_d = pltpu.make_async_copy(
        ei_hbm.at[1, pl.ds(base, E_PER_W)], didx_all, sem_s.at[1])
    ld_o.start()
    ld_d.start()
    pltpu.sync_copy(zeros_hbm, stg)
    pltpu.sync_copy(stg, acc_sh.at[pl.ds(s * RPS, RPS)])
    ld_o.wait()
    ld_d.wait()
    plsc.subcore_barrier()

    def scat_of(j, b):
        return pltpu.make_async_copy(
            ones, acc_sh.at[didx_all.at[pl.ds(j * CHUNK, CHUNK)]],
            sem_s.at[b])

    def fire(j, b):
        pltpu.async_copy(
            ones, acc_sh.at[didx_all.at[pl.ds(j * CHUNK, CHUNK)]],
            sem_s.at[b], add=True)

    for b in range(DEG_NBUF):
        fire(b, b)

    def outer(g, carry):
        j0 = g * DEG_NBUF
        for b in range(DEG_NBUF):
            scat_of(j0 + b, b).wait()
            fire(j0 + DEG_NBUF + b, b)
        return carry
    lax.fori_loop(0, DEG_NFULL - 1, outer, 0)

    j0 = (DEG_NFULL - 1) * DEG_NBUF
    for b in range(DEG_NREM):
        scat_of(j0 + b, b).wait()
        fire(DEG_NFULL * DEG_NBUF + b, b)
    for b in range(DEG_NREM):
        scat_of(DEG_NFULL * DEG_NBUF + b, b).wait()
    for b in range(DEG_NREM, DEG_NBUF):
        scat_of(j0 + b, b).wait()

    plsc.subcore_barrier()
    pltpu.sync_copy(acc_sh.at[pl.ds(s * RPS, RPS)], stg)
    pltpu.sync_copy(stg, out_hbm.at[c, pl.ds(s * RPS, RPS)])


def _lane_expand_consts():
    a = jnp.arange(128)[:, None]
    c = jnp.arange(128)[None, :]
    pe = (a == (c // HID) * DEG_W).astype(jnp.float32)
    po = (a == (c // HID) * DEG_W + 64).astype(jnp.float32)
    return pe, po


def _tcmm_body(x_ref, w1_ref, h_ref):
    w1 = w1_ref[...]
    parts = [
        jnp.dot(x_ref[:, k, :], w1, preferred_element_type=jnp.float32)
        for k in range(8)
    ]
    h_ref[...] = jnp.concatenate(parts, axis=1)


def _tcmm(x, W1):
    return pl.pallas_call(
        _tcmm_body,
        out_shape=jax.ShapeDtypeStruct((NW8, 128), jnp.float32),
    )(x.reshape(NW8, 8, IN_D), W1)


def _tca_body(h_ref, d8_ref, pe_ref, po_ref, g1_ref, dinv_ref):
    deg8 = d8_ref[0] + d8_ref[1] + 1.0
    dup = jnp.reshape(
        jnp.broadcast_to(deg8[:, None, :], (N // 16, 2, 128)), (NW8, 128))
    e = jnp.dot(dup, pe_ref[...], preferred_element_type=jnp.float32)
    o = jnp.dot(dup, po_ref[...], preferred_element_type=jnp.float32)
    par = (lax.broadcasted_iota(jnp.int32, (NW8, 128), 0) % 2) == 1
    dinv = lax.rsqrt(jnp.where(par, o, e))
    dinv_ref[...] = dinv
    g1_ref[...] = h_ref[...] * dinv


def _tca(h_w, deg8_w):
    pe, po = _lane_expand_consts()
    return pl.pallas_call(
        _tca_body,
        out_shape=[jax.ShapeDtypeStruct((NW8, 128), jnp.float32),
                   jax.ShapeDtypeStruct((NW8, 128), jnp.float32)],
    )(h_w, deg8_w, pe, po)


def _tcb_body(acc_ref, g1_ref, dinv_ref, b1_ref, g2_ref):
    dinv = dinv_ref[...]
    z = jnp.maximum(dinv * (acc_ref[0] + acc_ref[1] + g1_ref[...])
                    + b1_ref[...], 0.0)
    g2_ref[...] = dinv * z


def _tcb(acc1_w, g1_w, dinv_w, b1):
    return pl.pallas_call(
        _tcb_body,
        out_shape=jax.ShapeDtypeStruct((NW8, 128), jnp.float32),
    )(acc1_w, g1_w, dinv_w, jnp.tile(b1, 8).reshape(1, 128))


def _tcc_body(acc_ref, g2_ref, dinv_ref, w2b_ref, b2w_ref, savg_ref,
              ssum_ref, out_ref):
    t_w = dinv_ref[...] * (acc_ref[0] + acc_ref[1] + g2_ref[...])
    h = jnp.dot(t_w, w2b_ref[...], preferred_element_type=jnp.float32)
    h = h + b2w_ref[...]
    m = jnp.dot(h, savg_ref[...], preferred_element_type=jnp.float32)
    e = jnp.exp(h - m)
    lse = jnp.log(jnp.dot(e, ssum_ref[...],
                          preferred_element_type=jnp.float32))
    out_ref[...] = h - m - lse


def _tcc(acc2_w, g2_w, dinv_w, W2, b2):
    eye8 = jnp.eye(8, dtype=jnp.float32)
    w2big = jnp.kron(eye8, W2)
    savg = jnp.kron(eye8, jnp.full((OUT_D, OUT_D), 0.25, jnp.float32))
    ssum = jnp.kron(eye8, jnp.ones((OUT_D, OUT_D), jnp.float32))
    b2w = jnp.tile(b2, 8).reshape(1, 8 * OUT_D)
    out_w = pl.pallas_call(
        _tcc_body,
        out_shape=jax.ShapeDtypeStruct((NW8, 8 * OUT_D), jnp.float32),
    )(acc2_w, g2_w, dinv_w, w2big, b2w, savg, ssum)
    return out_w.reshape(N, OUT_D)


def kernel(x, edge_index, W1, b1, W2, b2):
    ei = edge_index.astype(jnp.int32)
    zeros_c = jnp.zeros((RPS, HID), jnp.float32)
    zeros8_c = jnp.zeros((RPS, DEG_W), jnp.float32)
    ones8_c = jnp.ones((CHUNK, DEG_W), jnp.float32)
    deg8_w = _deg_pass(ei, zeros8_c, ones8_c).reshape(NC, N // 16, 128)
    h_w = _tcmm(x, W1)
    g1_w, dinv_w = _tca(h_w, deg8_w)
    acc1_w = _msg_pass(g1_w.reshape(N, HID), ei, zeros_c).reshape(NC, NW8, 128)
    g2_w = _tcb(acc1_w, g1_w, dinv_w, b1)
    acc2_w = _msg_pass(g2_w.reshape(N, HID), ei, zeros_c).reshape(NC, NW8, 128)
    return _tcc(acc2_w, g2_w, dinv_w, W2, b2)
